# baseline (speedup 1.0000x reference)
import functools

import jax
import jax.numpy as jnp
from jax import lax
from jax.experimental import pallas as pl
from jax.experimental.pallas import tpu as pltpu
from jax.experimental.pallas import tpu_sc as plsc

_B = 16384
_D = 64

_NC, _NS = 2, 16
_NW = _NC * _NS
_BPW = _B // _NW
_CHUNK = 128
_NCHUNK = _BPW // _CHUNK


@functools.cache
def _sc_gather_fn():
    @functools.partial(
        pl.kernel,
        mesh=plsc.VectorSubcoreMesh(core_axis_name="c", subcore_axis_name="s",
                                    num_cores=_NC, num_subcores=_NS),
        out_type=(
            jax.ShapeDtypeStruct((_B, _D), jnp.float32),
            jax.ShapeDtypeStruct((_B, _D), jnp.float32),
        ),
        scratch_types=[
            pltpu.VMEM((_NCHUNK, _CHUNK), jnp.int32),
            pltpu.VMEM((_BPW, _D), jnp.float32),
            pltpu.VMEM((_NCHUNK, _CHUNK), jnp.int32),
            pltpu.VMEM((_BPW, _D), jnp.float32),
            pltpu.SemaphoreType.DMA,
        ],
        compiler_params=pltpu.CompilerParams(use_tc_tiling_on_sc=False),
    )
    def _sc_gather(user_hbm, item_hbm, utab_hbm, itab_hbm, p_hbm, q_hbm,
                   uidx_v, urows_v, iidx_v, irows_v, sem):
        wid = lax.axis_index("s") * _NC + lax.axis_index("c")
        base = wid * _BPW
        pltpu.sync_copy(user_hbm.at[wid], uidx_v)
        pltpu.sync_copy(item_hbm.at[wid], iidx_v)
        copies = []
        for j in range(_NCHUNK):
            copies.append(pltpu.async_copy(
                utab_hbm.at[uidx_v.at[j]],
                urows_v.at[pl.ds(j * _CHUNK, _CHUNK)], sem))
            copies.append(pltpu.async_copy(
                itab_hbm.at[iidx_v.at[j]],
                irows_v.at[pl.ds(j * _CHUNK, _CHUNK)], sem))
        for cp in copies:
            cp.wait()
        pltpu.sync_copy(urows_v, p_hbm.at[pl.ds(base, _BPW)])
        pltpu.sync_copy(irows_v, q_hbm.at[pl.ds(base, _BPW)])

    return _sc_gather


def _mlp_body(p_ref, q_ref, W1_ref, b1_ref, W2_ref, b2_ref, W3_ref, b3_ref,
              wl_ref, bl_ref, out_ref):
    p = p_ref[...]
    q = q_ref[...]
    x = jnp.concatenate((p, q), axis=-1)
    h = jnp.dot(x, W1_ref[...], preferred_element_type=jnp.float32) + b1_ref[...]
    h = jnp.where(h > 0, h, 0.01 * h)
    h = jnp.dot(h, W2_ref[...], preferred_element_type=jnp.float32) + b2_ref[...]
    h = jnp.where(h > 0, h, 0.01 * h)
    m = jnp.dot(h, W3_ref[...], preferred_element_type=jnp.float32) + b3_ref[...]
    mf = jnp.concatenate((p * q, m), axis=-1)
    out_ref[...] = (jnp.sum(mf * wl_ref[...], axis=-1, keepdims=True)
                    + bl_ref[...])


_BLK = 2048


def _mlp_call(p, q, W1, b1, W2, b2, W3, b3, wlT, bl):
    full = lambda shape: pl.BlockSpec(shape, lambda i: (0,) * len(shape))
    return pl.pallas_call(
        _mlp_body,
        grid=(_B // _BLK,),
        in_specs=[
            pl.BlockSpec((_BLK, _D), lambda i: (i, 0)),
            pl.BlockSpec((_BLK, _D), lambda i: (i, 0)),
            full((2 * _D, 256)),
            full((1, 256)),
            full((256, 256)),
            full((1, 256)),
            full((256, _D)),
            full((1, _D)),
            full((1, 2 * _D)),
            full((1, 1)),
        ],
        out_specs=pl.BlockSpec((_BLK, 1), lambda i: (i, 0)),
        out_shape=jax.ShapeDtypeStruct((_B, 1), jnp.float32),
    )(p, q, W1, b1, W2, b2, W3, b3, wlT, bl)


def kernel(user, item, user_table, item_table, W1, b1, W2, b2, W3, b3, Wl, bl):
    user_r = user.reshape(_NW, _NCHUNK, _CHUNK)
    item_r = item.reshape(_NW, _NCHUNK, _CHUNK)
    p, q = _sc_gather_fn()(user_r, item_r, user_table, item_table)
    return _mlp_call(p, q, W1, b1.reshape(1, -1), W2, b2.reshape(1, -1),
                     W3, b3.reshape(1, -1), Wl.reshape(1, 2 * _D),
                     bl.reshape(1, 1))


# speedup vs baseline: 1.0135x; 1.0135x over previous
import functools

import jax
import jax.numpy as jnp
from jax import lax
from jax.experimental import pallas as pl
from jax.experimental.pallas import tpu as pltpu
from jax.experimental.pallas import tpu_sc as plsc

_B = 16384
_D = 64
_W = 2 * _D

_NC, _NS = 2, 16
_NW = _NC * _NS
_BPW = _B // _NW
_CHUNK = 128
_NCHUNK = _BPW // _CHUNK


@functools.cache
def _sc_gather_fn():
    @functools.partial(
        pl.kernel,
        mesh=plsc.VectorSubcoreMesh(core_axis_name="c", subcore_axis_name="s",
                                    num_cores=_NC, num_subcores=_NS),
        out_type=(
            jax.ShapeDtypeStruct((_B, _W), jnp.float32),
            jax.ShapeDtypeStruct((_B, _W), jnp.float32),
        ),
        scratch_types=[
            pltpu.VMEM((_NCHUNK, _CHUNK), jnp.int32),
            pltpu.VMEM((_NCHUNK, _CHUNK), jnp.int32),
            pltpu.VMEM((_BPW, _W), jnp.float32),
            pltpu.SemaphoreType.DMA,
        ],
    )
    def _sc_gather(user_hbm, item_hbm, utab_hbm, itab_hbm, p_hbm, q_hbm,
                   uidx_v, iidx_v, rows_v, sem):
        wid = lax.axis_index("s") * _NC + lax.axis_index("c")
        base = wid * _BPW
        pltpu.sync_copy(user_hbm.at[wid], uidx_v)
        pltpu.sync_copy(item_hbm.at[wid], iidx_v)
        for tab_hbm, idx_v, out_hbm in ((utab_hbm, uidx_v, p_hbm),
                                        (itab_hbm, iidx_v, q_hbm)):
            copies = [
                pltpu.async_copy(tab_hbm.at[idx_v.at[j]],
                                 rows_v.at[pl.ds(j * _CHUNK, _CHUNK)], sem)
                for j in range(_NCHUNK)
            ]
            for cp in copies:
                cp.wait()
            pltpu.sync_copy(rows_v, out_hbm.at[pl.ds(base, _BPW)])

    return _sc_gather


def _mlp_body(gu_ref, gi_ref, up_ref, ip_ref, W1_ref, b1_ref, W2_ref, b2_ref,
              W3_ref, b3_ref, wl_ref, bl_ref, out_ref):
    gu = gu_ref[...]
    gi = gi_ref[...]
    p = jnp.where(up_ref[...] == 1, gu[:, _D:], gu[:, :_D])
    q = jnp.where(ip_ref[...] == 1, gi[:, _D:], gi[:, :_D])
    x = jnp.concatenate((p, q), axis=-1)
    h = jnp.dot(x, W1_ref[...], preferred_element_type=jnp.float32) + b1_ref[...]
    h = jnp.where(h > 0, h, 0.01 * h)
    h = jnp.dot(h, W2_ref[...], preferred_element_type=jnp.float32) + b2_ref[...]
    h = jnp.where(h > 0, h, 0.01 * h)
    m = jnp.dot(h, W3_ref[...], preferred_element_type=jnp.float32) + b3_ref[...]
    mf = jnp.concatenate((p * q, m), axis=-1)
    out_ref[...] = (jnp.sum(mf * wl_ref[...], axis=-1, keepdims=True)
                    + bl_ref[...])


_BLK = 2048


def _mlp_call(gu, gi, up, ip, W1, b1, W2, b2, W3, b3, wlT, bl):
    full = lambda shape: pl.BlockSpec(shape, lambda i: (0,) * len(shape))
    return pl.pallas_call(
        _mlp_body,
        grid=(_B // _BLK,),
        in_specs=[
            pl.BlockSpec((_BLK, _W), lambda i: (i, 0)),
            pl.BlockSpec((_BLK, _W), lambda i: (i, 0)),
            pl.BlockSpec((_BLK, 1), lambda i: (i, 0)),
            pl.BlockSpec((_BLK, 1), lambda i: (i, 0)),
            full((2 * _D, 256)),
            full((1, 256)),
            full((256, 256)),
            full((1, 256)),
            full((256, _D)),
            full((1, _D)),
            full((1, 2 * _D)),
            full((1, 1)),
        ],
        out_specs=pl.BlockSpec((_BLK, 1), lambda i: (i, 0)),
        out_shape=jax.ShapeDtypeStruct((_B, 1), jnp.float32),
    )(gu, gi, up, ip, W1, b1, W2, b2, W3, b3, wlT, bl)


def kernel(user, item, user_table, item_table, W1, b1, W2, b2, W3, b3, Wl, bl):
    tab_u = user_table.reshape(-1, _W)
    tab_i = item_table.reshape(-1, _W)
    u2 = (user >> 1).reshape(_NW, _NCHUNK, _CHUNK)
    i2 = (item >> 1).reshape(_NW, _NCHUNK, _CHUNK)
    gu, gi = _sc_gather_fn()(u2, i2, tab_u, tab_i)
    up = (user & 1).reshape(_B, 1)
    ip = (item & 1).reshape(_B, 1)
    return _mlp_call(gu, gi, up, ip, W1, b1.reshape(1, -1), W2,
                     b2.reshape(1, -1), W3, b3.reshape(1, -1),
                     Wl.reshape(1, 2 * _D), bl.reshape(1, 1))
